# baseline (speedup 1.0000x reference)
import functools
import math

import jax
import jax.numpy as jnp
from jax import lax
from jax.experimental import pallas as pl
from jax.experimental.pallas import tpu as pltpu
from jax.experimental.pallas import tpu_sc as plsc

_NC = 2
_NS = 16
_NW = _NC * _NS
_LANES = 16
_BBLK = 128


def _make_embed(batch: int, seq: int, vocab: int, d: int):
    assert batch == _NW * _BBLK
    assert seq % 2 == 0 and d % 8 == 0
    n_pairs = seq // 2
    rows_per_w = _BBLK * seq
    scale = jnp.float32(math.sqrt(d))
    d8 = d // 8
    cgrp = _BBLK // _LANES
    mesh = plsc.VectorSubcoreMesh(core_axis_name="c", subcore_axis_name="s")

    @functools.partial(
        pl.kernel,
        mesh=mesh,
        out_type=jax.ShapeDtypeStruct((seq, d8, _NW, 8, _BBLK), jnp.float32),
        scratch_types=[
            pltpu.VMEM((rows_per_w,), jnp.int32),
            pltpu.VMEM((_BBLK,), jnp.int32),
            pltpu.VMEM((_BBLK,), jnp.int32),
            pltpu.VMEM((_BBLK, d), jnp.float32),
            pltpu.VMEM((_BBLK, d), jnp.float32),
            pltpu.VMEM((d8, 8, _BBLK), jnp.float32),
            pltpu.SemaphoreType.DMA,
            pltpu.SemaphoreType.DMA,
        ],
        compiler_params=pltpu.CompilerParams(
            use_tc_tiling_on_sc=False, needs_layout_passes=False
        ),
    )
    def embed(idx_hbm, table_hbm, out_hbm, idx_all, iblk0, iblk1, rows0, rows1,
              tbuf, sem0, sem1):
        wid = lax.axis_index("s") * _NC + lax.axis_index("c")
        pltpu.sync_copy(idx_hbm.at[pl.ds(wid * rows_per_w, rows_per_w)], idx_all)
        lanes = lax.iota(jnp.int32, 16)
        lane_base = lanes * seq

        def build_iblk(s, iblk):
            for c8 in range(cgrp):
                pos = lane_base + (c8 * _LANES * seq + s)
                iblk[pl.ds(c8 * _LANES, _LANES)] = plsc.load_gather(idx_all, [pos])

        def start_gather(iblk, rows_v, sem):
            pltpu.async_copy(table_hbm.at[iblk], rows_v, sem)

        def finish_block(s, iblk, rows_v, sem):
            pltpu.make_async_copy(table_hbm.at[iblk], rows_v, sem).wait()

            @plsc.parallel_loop(0, d8, step=1)
            def tr_body(tr):
                for r in range(8):
                    col = jnp.full((16,), tr * 8 + r, jnp.int32)
                    for c8 in range(cgrp):
                        row = lanes + c8 * _LANES
                        v = plsc.load_gather(rows_v, [row, col])
                        tbuf[tr, r, pl.ds(c8 * _LANES, _LANES)] = v * scale

            pltpu.sync_copy(tbuf, out_hbm.at[s, :, wid])

        build_iblk(0, iblk0)
        start_gather(iblk0, rows0, sem0)

        def pair_body(p, carry):
            s = 2 * p
            build_iblk(s + 1, iblk1)
            start_gather(iblk1, rows1, sem1)
            finish_block(s, iblk0, rows0, sem0)

            @pl.when(p + 1 < n_pairs)
            def _():
                build_iblk(s + 2, iblk0)
                start_gather(iblk0, rows0, sem0)

            finish_block(s + 1, iblk1, rows1, sem1)
            return carry

        lax.fori_loop(0, n_pairs, pair_body, 0)

    return embed


def kernel(x, table):
    b, s = x.shape
    vocab, d = table.shape
    embed = _make_embed(b, s, vocab, d)
    out5 = embed(x.reshape(b * s), table)
    return out5.transpose(2, 4, 0, 1, 3).reshape(b, s, d)
